# initial kernel scaffold (unmeasured)
import jax
import jax.numpy as jnp
from jax import lax
from jax.experimental import pallas as pl
from jax.experimental.pallas import tpu as pltpu

N_DEV = 16
NSLOTS = 2


def kernel(x, w_mat):
    m, k_shard = x.shape
    _, n_out = w_mat.shape
    chunk = m // N_DEV

    def rows(c):
        return pl.ds(c * chunk, chunk)

    def body(x_ref, w_ref, out_ref, comm_ref,
             rs_send, rs_recv, ag_send, ag_recv, credit_rs, credit_ag):
        me = lax.axis_index("i")
        left = lax.rem(me + (N_DEV - 1), N_DEV)
        right = lax.rem(me + 1, N_DEV)

        barrier_sem = pltpu.get_barrier_semaphore()
        for nbr in (left, right):
            pl.semaphore_signal(
                barrier_sem, inc=1,
                device_id=(nbr,), device_id_type=pl.DeviceIdType.MESH,
            )
        pl.semaphore_wait(barrier_sem, 2)

        out_ref[...] = jnp.dot(
            x_ref[...], w_ref[...], preferred_element_type=jnp.float32
        )

        for s in range(N_DEV - 1):
            slot = s % NSLOTS
            c_send = lax.rem(me + (N_DEV - s), N_DEV)
            c_recv = lax.rem(me + (N_DEV - s - 1), N_DEV)
            if s >= NSLOTS:
                pl.semaphore_wait(credit_rs, 1)
            rdma = pltpu.make_async_remote_copy(
                src_ref=out_ref.at[rows(c_send)],
                dst_ref=comm_ref.at[slot],
                send_sem=rs_send.at[slot],
                recv_sem=rs_recv.at[slot],
                device_id=(right,),
                device_id_type=pl.DeviceIdType.MESH,
            )
            rdma.start()
            rdma.wait()
            out_ref[rows(c_recv)] = out_ref[rows(c_recv)] + comm_ref[slot]
            if s + NSLOTS <= N_DEV - 2:
                pl.semaphore_signal(
                    credit_rs, inc=1,
                    device_id=(left,), device_id_type=pl.DeviceIdType.MESH,
                )

        c_own = lax.rem(me + 1, N_DEV)
        out_ref[rows(c_own)] = jnp.maximum(out_ref[rows(c_own)], 0.0)

        for s in range(N_DEV - 1):
            slot = s % NSLOTS
            c_send = lax.rem(me + (N_DEV + 1 - s), N_DEV)
            if s >= NSLOTS:
                pl.semaphore_wait(credit_ag, 1)
            rdma = pltpu.make_async_remote_copy(
                src_ref=out_ref.at[rows(c_send)],
                dst_ref=out_ref.at[rows(c_send)],
                send_sem=ag_send.at[slot],
                recv_sem=ag_recv.at[slot],
                device_id=(right,),
                device_id_type=pl.DeviceIdType.MESH,
            )
            rdma.start()
            rdma.wait()
            if s + NSLOTS <= N_DEV - 2:
                pl.semaphore_signal(
                    credit_ag, inc=1,
                    device_id=(left,), device_id_type=pl.DeviceIdType.MESH,
                )

    return pl.pallas_call(
        body,
        out_shape=jax.ShapeDtypeStruct((m, n_out), jnp.float32),
        in_specs=[
            pl.BlockSpec(memory_space=pltpu.VMEM),
            pl.BlockSpec(memory_space=pltpu.VMEM),
        ],
        out_specs=pl.BlockSpec(memory_space=pltpu.VMEM),
        scratch_shapes=[
            pltpu.VMEM((NSLOTS, chunk, n_out), jnp.float32),
            pltpu.SemaphoreType.DMA((NSLOTS,)),
            pltpu.SemaphoreType.DMA((NSLOTS,)),
            pltpu.SemaphoreType.DMA((NSLOTS,)),
            pltpu.SemaphoreType.DMA((NSLOTS,)),
            pltpu.SemaphoreType.REGULAR,
            pltpu.SemaphoreType.REGULAR,
        ],
        compiler_params=pltpu.CompilerParams(collective_id=0),
    )(x, w_mat)


# baseline (device time: 777827 ns/iter reference)
import jax
import jax.numpy as jnp
from jax import lax
from jax.experimental import pallas as pl
from jax.experimental.pallas import tpu as pltpu

N_DEV = 16
NSLOTS = 2


def kernel(x, w_mat):
    m, k_shard = x.shape
    _, n_out = w_mat.shape
    chunk = m // N_DEV

    def rows(c):
        return pl.ds(c * chunk, chunk)

    def body(x_ref, w_ref, out_ref, comm_ref,
             rs_send, rs_recv, ag_send, ag_recv, credit_rs, credit_ag):
        me = lax.axis_index("i")
        left = lax.rem(me + (N_DEV - 1), N_DEV)
        right = lax.rem(me + 1, N_DEV)

        barrier_sem = pltpu.get_barrier_semaphore()
        for nbr in (left, right):
            pl.semaphore_signal(
                barrier_sem, inc=1,
                device_id=(nbr,), device_id_type=pl.DeviceIdType.MESH,
            )
        pl.semaphore_wait(barrier_sem, 2)

        out_ref[...] = jnp.dot(
            x_ref[...], w_ref[...], preferred_element_type=jnp.float32
        )

        for s in range(N_DEV - 1):
            slot = s % NSLOTS
            c_send = lax.rem(me + (N_DEV - s), N_DEV)
            c_recv = lax.rem(me + (N_DEV - s - 1), N_DEV)
            if s >= NSLOTS:
                pl.semaphore_wait(credit_rs, 1)
            rdma = pltpu.make_async_remote_copy(
                src_ref=out_ref.at[rows(c_send)],
                dst_ref=comm_ref.at[slot],
                send_sem=rs_send.at[slot],
                recv_sem=rs_recv.at[slot],
                device_id=(right,),
                device_id_type=pl.DeviceIdType.MESH,
            )
            rdma.start()
            rdma.wait()
            out_ref[rows(c_recv)] = out_ref[rows(c_recv)] + comm_ref[slot]
            if s + NSLOTS <= N_DEV - 2:
                pl.semaphore_signal(
                    credit_rs, inc=1,
                    device_id=(left,), device_id_type=pl.DeviceIdType.MESH,
                )

        c_own = lax.rem(me + 1, N_DEV)
        out_ref[rows(c_own)] = jnp.maximum(out_ref[rows(c_own)], 0.0)

        for s in range(N_DEV - 1):
            slot = s % NSLOTS
            c_send = lax.rem(me + (N_DEV + 1 - s), N_DEV)
            if s >= NSLOTS:
                pl.semaphore_wait(credit_ag, 1)
            rdma = pltpu.make_async_remote_copy(
                src_ref=out_ref.at[rows(c_send)],
                dst_ref=out_ref.at[rows(c_send)],
                send_sem=ag_send.at[slot],
                recv_sem=ag_recv.at[slot],
                device_id=(right,),
                device_id_type=pl.DeviceIdType.MESH,
            )
            rdma.start()
            rdma.wait()
            if s + NSLOTS <= N_DEV - 2:
                pl.semaphore_signal(
                    credit_ag, inc=1,
                    device_id=(left,), device_id_type=pl.DeviceIdType.MESH,
                )

    return pl.pallas_call(
        body,
        out_shape=jax.ShapeDtypeStruct((m, n_out), jnp.float32),
        in_specs=[
            pl.BlockSpec(memory_space=pltpu.VMEM),
            pl.BlockSpec(memory_space=pltpu.VMEM),
        ],
        out_specs=pl.BlockSpec(memory_space=pltpu.VMEM),
        scratch_shapes=[
            pltpu.VMEM((NSLOTS, chunk, n_out), jnp.float32),
            pltpu.SemaphoreType.DMA((NSLOTS,)),
            pltpu.SemaphoreType.DMA((NSLOTS,)),
            pltpu.SemaphoreType.DMA((NSLOTS,)),
            pltpu.SemaphoreType.DMA((NSLOTS,)),
            pltpu.SemaphoreType.REGULAR,
            pltpu.SemaphoreType.REGULAR,
        ],
        compiler_params=pltpu.CompilerParams(
            collective_id=0,
            vmem_limit_bytes=56 * 1024 * 1024,
        ),
    )(x, w_mat)


# device time: 480734 ns/iter; 1.6180x vs baseline; 1.6180x over previous
import jax
import jax.numpy as jnp
from jax import lax
from jax.experimental import pallas as pl
from jax.experimental.pallas import tpu as pltpu

N_DEV = 16
NSLOTS = 2


def kernel(x, w_mat):
    m, k_shard = x.shape
    _, n_out = w_mat.shape
    chunk = m // N_DEV
    half = n_out // 2

    def rows(c):
        return pl.ds(c * chunk, chunk)

    cw_cols = pl.ds(0, half)
    ccw_cols = pl.ds(half, half)

    def body(x_ref, w_ref, out_ref, comm_cw, comm_ccw,
             rs_send_cw, rs_recv_cw, rs_send_ccw, rs_recv_ccw,
             ag_send_cw, ag_recv_cw, ag_send_ccw, ag_recv_ccw,
             credit_rs_cw, credit_rs_ccw, credit_ag_cw, credit_ag_ccw):
        me = lax.axis_index("i")
        left = lax.rem(me + (N_DEV - 1), N_DEV)
        right = lax.rem(me + 1, N_DEV)

        barrier_sem = pltpu.get_barrier_semaphore()
        for nbr in (left, right):
            pl.semaphore_signal(
                barrier_sem, inc=1,
                device_id=(nbr,), device_id_type=pl.DeviceIdType.MESH,
            )
        pl.semaphore_wait(barrier_sem, 2)

        out_ref[...] = jnp.dot(
            x_ref[...], w_ref[...], preferred_element_type=jnp.float32
        )

        for s in range(N_DEV - 1):
            slot = s % NSLOTS
            if s >= NSLOTS:
                pl.semaphore_wait(credit_rs_cw, 1)
                pl.semaphore_wait(credit_rs_ccw, 1)
            cw = pltpu.make_async_remote_copy(
                src_ref=out_ref.at[rows(lax.rem(me + (N_DEV - s), N_DEV)), cw_cols],
                dst_ref=comm_cw.at[slot],
                send_sem=rs_send_cw.at[slot],
                recv_sem=rs_recv_cw.at[slot],
                device_id=(right,),
                device_id_type=pl.DeviceIdType.MESH,
            )
            ccw = pltpu.make_async_remote_copy(
                src_ref=out_ref.at[rows(lax.rem(me + s, N_DEV)), ccw_cols],
                dst_ref=comm_ccw.at[slot],
                send_sem=rs_send_ccw.at[slot],
                recv_sem=rs_recv_ccw.at[slot],
                device_id=(left,),
                device_id_type=pl.DeviceIdType.MESH,
            )
            cw.start()
            ccw.start()
            cw.wait()
            ccw.wait()
            r_cw = rows(lax.rem(me + (N_DEV - s - 1), N_DEV))
            r_ccw = rows(lax.rem(me + s + 1, N_DEV))
            out_ref[r_cw, cw_cols] = out_ref[r_cw, cw_cols] + comm_cw[slot]
            out_ref[r_ccw, ccw_cols] = out_ref[r_ccw, ccw_cols] + comm_ccw[slot]
            if s + NSLOTS <= N_DEV - 2:
                pl.semaphore_signal(
                    credit_rs_cw, inc=1,
                    device_id=(left,), device_id_type=pl.DeviceIdType.MESH,
                )
                pl.semaphore_signal(
                    credit_rs_ccw, inc=1,
                    device_id=(right,), device_id_type=pl.DeviceIdType.MESH,
                )

        r_own_cw = rows(lax.rem(me + 1, N_DEV))
        r_own_ccw = rows(lax.rem(me + (N_DEV - 1), N_DEV))
        out_ref[r_own_cw, cw_cols] = jnp.maximum(out_ref[r_own_cw, cw_cols], 0.0)
        out_ref[r_own_ccw, ccw_cols] = jnp.maximum(
            out_ref[r_own_ccw, ccw_cols], 0.0
        )

        for s in range(N_DEV - 1):
            slot = s % NSLOTS
            if s >= NSLOTS:
                pl.semaphore_wait(credit_ag_cw, 1)
                pl.semaphore_wait(credit_ag_ccw, 1)
            c_cw = lax.rem(me + (N_DEV + 1 - s), N_DEV)
            c_ccw = lax.rem(me + (N_DEV - 1 + s), N_DEV)
            cw = pltpu.make_async_remote_copy(
                src_ref=out_ref.at[rows(c_cw), cw_cols],
                dst_ref=out_ref.at[rows(c_cw), cw_cols],
                send_sem=ag_send_cw.at[slot],
                recv_sem=ag_recv_cw.at[slot],
                device_id=(right,),
                device_id_type=pl.DeviceIdType.MESH,
            )
            ccw = pltpu.make_async_remote_copy(
                src_ref=out_ref.at[rows(c_ccw), ccw_cols],
                dst_ref=out_ref.at[rows(c_ccw), ccw_cols],
                send_sem=ag_send_ccw.at[slot],
                recv_sem=ag_recv_ccw.at[slot],
                device_id=(left,),
                device_id_type=pl.DeviceIdType.MESH,
            )
            cw.start()
            ccw.start()
            cw.wait()
            ccw.wait()
            if s + NSLOTS <= N_DEV - 2:
                pl.semaphore_signal(
                    credit_ag_cw, inc=1,
                    device_id=(left,), device_id_type=pl.DeviceIdType.MESH,
                )
                pl.semaphore_signal(
                    credit_ag_ccw, inc=1,
                    device_id=(right,), device_id_type=pl.DeviceIdType.MESH,
                )

    return pl.pallas_call(
        body,
        out_shape=jax.ShapeDtypeStruct((m, n_out), jnp.float32),
        in_specs=[
            pl.BlockSpec(memory_space=pltpu.VMEM),
            pl.BlockSpec(memory_space=pltpu.VMEM),
        ],
        out_specs=pl.BlockSpec(memory_space=pltpu.VMEM),
        scratch_shapes=[
            pltpu.VMEM((NSLOTS, chunk, half), jnp.float32),
            pltpu.VMEM((NSLOTS, chunk, half), jnp.float32),
            pltpu.SemaphoreType.DMA((NSLOTS,)),
            pltpu.SemaphoreType.DMA((NSLOTS,)),
            pltpu.SemaphoreType.DMA((NSLOTS,)),
            pltpu.SemaphoreType.DMA((NSLOTS,)),
            pltpu.SemaphoreType.DMA((NSLOTS,)),
            pltpu.SemaphoreType.DMA((NSLOTS,)),
            pltpu.SemaphoreType.DMA((NSLOTS,)),
            pltpu.SemaphoreType.DMA((NSLOTS,)),
            pltpu.SemaphoreType.REGULAR,
            pltpu.SemaphoreType.REGULAR,
            pltpu.SemaphoreType.REGULAR,
            pltpu.SemaphoreType.REGULAR,
        ],
        compiler_params=pltpu.CompilerParams(
            collective_id=0,
            vmem_limit_bytes=56 * 1024 * 1024,
        ),
    )(x, w_mat)


# device time: 394747 ns/iter; 1.9704x vs baseline; 1.2178x over previous
import jax
import jax.numpy as jnp
from jax import lax
from jax.experimental import pallas as pl
from jax.experimental.pallas import tpu as pltpu

N_DEV = 16
NSLOTS = 2
NSUB = 4
LAST = N_DEV - 2


def kernel(x, w_mat):
    m, k_shard = x.shape
    _, n_out = w_mat.shape
    chunk = m // N_DEV
    subw = n_out // NSUB

    def rows(c):
        return pl.ds(c * chunk, chunk)

    def cols(r):
        return pl.ds(r * subw, subw)

    def body(x_ref, w_ref, out_ref, comm,
             rs_send, rs_recv, ag_send, ag_recv, credit_rs, credit_ag):
        me = lax.axis_index("i")
        left = lax.rem(me + (N_DEV - 1), N_DEV)
        right = lax.rem(me + 1, N_DEV)

        barrier_sem = pltpu.get_barrier_semaphore()
        for nbr in (left, right):
            pl.semaphore_signal(
                barrier_sem, inc=1,
                device_id=(nbr,), device_id_type=pl.DeviceIdType.MESH,
            )
        pl.semaphore_wait(barrier_sem, 2)

        out_ref[...] = jnp.dot(
            x_ref[...], w_ref[...], preferred_element_type=jnp.float32
        )

        def dst_dev(r):
            return right if r < 2 else left

        def src_dev(r):
            return left if r < 2 else right

        def rs_send_chunk(r, s):
            return lax.rem(me + (N_DEV - s if r < 2 else s), N_DEV)

        def rs_recv_chunk(r, s):
            return lax.rem(me + (N_DEV - s - 1 if r < 2 else s + 1), N_DEV)

        def rs_desc(r, s):
            slot = s % NSLOTS
            return pltpu.make_async_remote_copy(
                src_ref=out_ref.at[rows(rs_send_chunk(r, s)), cols(r)],
                dst_ref=comm.at[r, slot],
                send_sem=rs_send.at[r, slot],
                recv_sem=rs_recv.at[r, slot],
                device_id=(dst_dev(r),),
                device_id_type=pl.DeviceIdType.MESH,
            )

        for r in range(NSUB):
            rs_desc(r, 0).start()
        for s in range(N_DEV - 1):
            for r in range(NSUB):
                rs_desc(r, s).wait_recv()
                rc = rows(rs_recv_chunk(r, s))
                out_ref[rc, cols(r)] = out_ref[rc, cols(r)] + comm[r, s % NSLOTS]
                if s + NSLOTS <= LAST:
                    pl.semaphore_signal(
                        credit_rs.at[r], inc=1,
                        device_id=(src_dev(r),),
                        device_id_type=pl.DeviceIdType.MESH,
                    )
                if s + 1 <= LAST:
                    if s + 1 >= NSLOTS:
                        pl.semaphore_wait(credit_rs.at[r], 1)
                    if s >= 1:
                        rs_desc(r, s - 1).wait_send()
                    rs_desc(r, s + 1).start()
        for r in range(NSUB):
            rs_desc(r, LAST).wait_send()
            rs_desc(r, LAST + 1).wait_send()

        for r in range(NSUB):
            ro = rows(lax.rem(me + (1 if r < 2 else N_DEV - 1), N_DEV))
            out_ref[ro, cols(r)] = jnp.maximum(out_ref[ro, cols(r)], 0.0)

        def ag_chunk(r, s):
            return lax.rem(me + (N_DEV + 1 - s if r < 2 else N_DEV - 1 + s), N_DEV)

        def ag_desc(r, s):
            slot = s % NSLOTS
            target = out_ref.at[rows(ag_chunk(r, s)), cols(r)]
            return pltpu.make_async_remote_copy(
                src_ref=target,
                dst_ref=target,
                send_sem=ag_send.at[r, slot],
                recv_sem=ag_recv.at[r, slot],
                device_id=(dst_dev(r),),
                device_id_type=pl.DeviceIdType.MESH,
            )

        for r in range(NSUB):
            ag_desc(r, 0).start()
        for s in range(N_DEV - 1):
            for r in range(NSUB):
                ag_desc(r, s).wait_recv()
                if s + NSLOTS <= LAST:
                    pl.semaphore_signal(
                        credit_ag.at[r], inc=1,
                        device_id=(src_dev(r),),
                        device_id_type=pl.DeviceIdType.MESH,
                    )
                if s + 1 <= LAST:
                    if s + 1 >= NSLOTS:
                        pl.semaphore_wait(credit_ag.at[r], 1)
                    if s >= 1:
                        ag_desc(r, s - 1).wait_send()
                    ag_desc(r, s + 1).start()
        for r in range(NSUB):
            ag_desc(r, LAST).wait_send()
            ag_desc(r, LAST + 1).wait_send()

    return pl.pallas_call(
        body,
        out_shape=jax.ShapeDtypeStruct((m, n_out), jnp.float32),
        in_specs=[
            pl.BlockSpec(memory_space=pltpu.VMEM),
            pl.BlockSpec(memory_space=pltpu.VMEM),
        ],
        out_specs=pl.BlockSpec(memory_space=pltpu.VMEM),
        scratch_shapes=[
            pltpu.VMEM((NSUB, NSLOTS, chunk, subw), jnp.float32),
            pltpu.SemaphoreType.DMA((NSUB, NSLOTS)),
            pltpu.SemaphoreType.DMA((NSUB, NSLOTS)),
            pltpu.SemaphoreType.DMA((NSUB, NSLOTS)),
            pltpu.SemaphoreType.DMA((NSUB, NSLOTS)),
            pltpu.SemaphoreType.REGULAR((NSUB,)),
            pltpu.SemaphoreType.REGULAR((NSUB,)),
        ],
        compiler_params=pltpu.CompilerParams(
            collective_id=0,
            vmem_limit_bytes=56 * 1024 * 1024,
        ),
    )(x, w_mat)


# device time: 391045 ns/iter; 1.9891x vs baseline; 1.0095x over previous
import jax
import jax.numpy as jnp
from jax import lax
from jax.experimental import pallas as pl
from jax.experimental.pallas import tpu as pltpu

N_DEV = 16
NSLOTS = 3
NSUB = 4
LAST = N_DEV - 2


def kernel(x, w_mat):
    m, k_shard = x.shape
    _, n_out = w_mat.shape
    chunk = m // N_DEV
    subw = n_out // NSUB

    def rows(c):
        return pl.ds(c * chunk, chunk)

    def cols(r):
        return pl.ds(r * subw, subw)

    def body(x_ref, w_ref, out_ref, comm,
             rs_send, rs_recv, ag_send, ag_recv, credit_rs, credit_ag):
        me = lax.axis_index("i")
        left = lax.rem(me + (N_DEV - 1), N_DEV)
        right = lax.rem(me + 1, N_DEV)

        barrier_sem = pltpu.get_barrier_semaphore()
        for nbr in (left, right):
            pl.semaphore_signal(
                barrier_sem, inc=1,
                device_id=(nbr,), device_id_type=pl.DeviceIdType.MESH,
            )
        pl.semaphore_wait(barrier_sem, 2)

        def gemm_chunk(c):
            rc = rows(c)
            out_ref[rc, :] = jnp.dot(
                x_ref[rc, :], w_ref[...], preferred_element_type=jnp.float32
            )

        gemm_chunk(me)
        gemm_chunk(lax.rem(me + 1, N_DEV))
        gemm_chunk(lax.rem(me + (N_DEV - 1), N_DEV))

        def dst_dev(r):
            return right if r < 2 else left

        def src_dev(r):
            return left if r < 2 else right

        def rs_send_chunk(r, s):
            return lax.rem(me + (N_DEV - s if r < 2 else s), N_DEV)

        def rs_recv_chunk(r, s):
            return lax.rem(me + (N_DEV - s - 1 if r < 2 else s + 1), N_DEV)

        def rs_desc(r, s):
            slot = s % NSLOTS
            return pltpu.make_async_remote_copy(
                src_ref=out_ref.at[rows(rs_send_chunk(r, s)), cols(r)],
                dst_ref=comm.at[r, slot],
                send_sem=rs_send.at[r, slot],
                recv_sem=rs_recv.at[r, slot],
                device_id=(dst_dev(r),),
                device_id_type=pl.DeviceIdType.MESH,
            )

        for r in range(NSUB):
            rs_desc(r, 0).start()
        for s in range(N_DEV - 1):
            for r in range(NSUB):
                rs_desc(r, s).wait_recv()
                rc = rows(rs_recv_chunk(r, s))
                out_ref[rc, cols(r)] = out_ref[rc, cols(r)] + comm[r, s % NSLOTS]
                if s + NSLOTS <= LAST:
                    pl.semaphore_signal(
                        credit_rs.at[r], inc=1,
                        device_id=(src_dev(r),),
                        device_id_type=pl.DeviceIdType.MESH,
                    )
                if s + 1 <= LAST:
                    if s + 1 >= NSLOTS:
                        pl.semaphore_wait(credit_rs.at[r], 1)
                        rs_desc(r, s + 1 - NSLOTS).wait_send()
                    rs_desc(r, s + 1).start()
            if s + 2 <= N_DEV // 2:
                gemm_chunk(lax.rem(me + (N_DEV - s - 2), N_DEV))
                if s + 2 < N_DEV // 2:
                    gemm_chunk(lax.rem(me + s + 2, N_DEV))
        for r in range(NSUB):
            for k in range(min(NSLOTS, N_DEV - 1)):
                rs_desc(r, LAST + 1 - k).wait_send()

        for r in range(NSUB):
            ro = rows(lax.rem(me + (1 if r < 2 else N_DEV - 1), N_DEV))
            out_ref[ro, cols(r)] = jnp.maximum(out_ref[ro, cols(r)], 0.0)

        def ag_chunk(r, s):
            return lax.rem(me + (N_DEV + 1 - s if r < 2 else N_DEV - 1 + s), N_DEV)

        def ag_desc(r, s):
            slot = s % NSLOTS
            target = out_ref.at[rows(ag_chunk(r, s)), cols(r)]
            return pltpu.make_async_remote_copy(
                src_ref=target,
                dst_ref=target,
                send_sem=ag_send.at[r, slot],
                recv_sem=ag_recv.at[r, slot],
                device_id=(dst_dev(r),),
                device_id_type=pl.DeviceIdType.MESH,
            )

        for r in range(NSUB):
            ag_desc(r, 0).start()
        for s in range(N_DEV - 1):
            for r in range(NSUB):
                ag_desc(r, s).wait_recv()
                if s + NSLOTS <= LAST:
                    pl.semaphore_signal(
                        credit_ag.at[r], inc=1,
                        device_id=(src_dev(r),),
                        device_id_type=pl.DeviceIdType.MESH,
                    )
                if s + 1 <= LAST:
                    if s + 1 >= NSLOTS:
                        pl.semaphore_wait(credit_ag.at[r], 1)
                        ag_desc(r, s + 1 - NSLOTS).wait_send()
                    ag_desc(r, s + 1).start()
        for r in range(NSUB):
            for k in range(min(NSLOTS, N_DEV - 1)):
                ag_desc(r, LAST + 1 - k).wait_send()

    return pl.pallas_call(
        body,
        out_shape=jax.ShapeDtypeStruct((m, n_out), jnp.float32),
        in_specs=[
            pl.BlockSpec(memory_space=pltpu.VMEM),
            pl.BlockSpec(memory_space=pltpu.VMEM),
        ],
        out_specs=pl.BlockSpec(memory_space=pltpu.VMEM),
        scratch_shapes=[
            pltpu.VMEM((NSUB, NSLOTS, chunk, subw), jnp.float32),
            pltpu.SemaphoreType.DMA((NSUB, NSLOTS)),
            pltpu.SemaphoreType.DMA((NSUB, NSLOTS)),
            pltpu.SemaphoreType.DMA((NSUB, NSLOTS)),
            pltpu.SemaphoreType.DMA((NSUB, NSLOTS)),
            pltpu.SemaphoreType.REGULAR((NSUB,)),
            pltpu.SemaphoreType.REGULAR((NSUB,)),
        ],
        compiler_params=pltpu.CompilerParams(
            collective_id=0,
            vmem_limit_bytes=56 * 1024 * 1024,
        ),
    )(x, w_mat)
